# Initial kernel scaffold; baseline (speedup 1.0000x reference)
import jax
import jax.numpy as jnp
from jax.experimental import pallas as pl


def kernel(src_token_ids_batch, tgt_token_ids_batch, src_table, tgt_table):
    raise NotImplementedError("write your pallas kernel here")


# speedup vs baseline: 1.0709x; 1.0709x over previous
import functools
import math

import jax
import jax.numpy as jnp
from jax import lax
from jax.experimental import pallas as pl
from jax.experimental.pallas import tpu as pltpu, tpu_sc as plsc

EMBED_DIM = 128
SEQ = 2048
BATCH = 4
SCALE = math.sqrt(EMBED_DIM)

NW = 32
ROWS = 8192
CHUNK = 128
CPW = (ROWS // NW) // CHUNK
L = 16


def _pe_table():
    position_id = jnp.arange(0, SEQ, dtype=jnp.float32)[:, None]
    frequencies = jnp.power(
        10000.0, -jnp.arange(0, EMBED_DIM, 2, dtype=jnp.float32) / EMBED_DIM)
    sin_part = jnp.sin(position_id * frequencies)
    cos_part = jnp.cos(position_id * frequencies)
    return jnp.stack([sin_part, cos_part], axis=-1).reshape(SEQ, EMBED_DIM)


@functools.partial(
    pl.kernel,
    mesh=plsc.VectorSubcoreMesh(core_axis_name="c", subcore_axis_name="s"),
    out_type=(
        jax.ShapeDtypeStruct((ROWS // CHUNK, CHUNK, EMBED_DIM), jnp.float32),
        jax.ShapeDtypeStruct((ROWS // CHUNK, CHUNK, EMBED_DIM), jnp.float32),
    ),
    scratch_types=[
        pltpu.VMEM((CPW, CHUNK), jnp.int32),
        pltpu.VMEM((CPW, CHUNK), jnp.int32),
        pltpu.VMEM((CPW, CHUNK, EMBED_DIM), jnp.float32),
        pltpu.VMEM((CPW, CHUNK, EMBED_DIM), jnp.float32),
        pltpu.VMEM((CPW, CHUNK, EMBED_DIM), jnp.float32),
        pltpu.SemaphoreType.DMA,
        pltpu.SemaphoreType.DMA,
    ],
)
def _sc_embed(src_ids, tgt_ids, src_tab, tgt_tab, pe,
              src_out, tgt_out,
              idx_s, idx_t, pe_v, rows_s, rows_t, sem_s, sem_t):
    cid = lax.axis_index("c")
    sid = lax.axis_index("s")
    wid = sid * 2 + cid
    row0 = wid * CPW
    prow = lax.rem(wid, 8) * CPW

    pltpu.sync_copy(src_ids.at[pl.ds(row0, CPW)], idx_s)
    pltpu.sync_copy(tgt_ids.at[pl.ds(row0, CPW)], idx_t)
    pltpu.sync_copy(pe.at[pl.ds(prow, CPW)], pe_v)

    cps = [pltpu.async_copy(src_tab.at[idx_s.at[j]], rows_s.at[j], sem_s)
           for j in range(CPW)]
    cpt = [pltpu.async_copy(tgt_tab.at[idx_t.at[j]], rows_t.at[j], sem_t)
           for j in range(CPW)]

    def fma_pass(rows_ref):
        def body(r, _):
            for j in range(CPW):
                for l in range(EMBED_DIM // L):
                    sl = pl.ds(l * L, L)
                    rows_ref[j, r, sl] = rows_ref[j, r, sl] * SCALE + pe_v[j, r, sl]
            return 0
        lax.fori_loop(0, CHUNK, body, 0)

    for cp in cps:
        cp.wait()
    fma_pass(rows_s)
    pltpu.sync_copy(rows_s, src_out.at[pl.ds(row0, CPW)])

    for cp in cpt:
        cp.wait()
    fma_pass(rows_t)
    pltpu.sync_copy(rows_t, tgt_out.at[pl.ds(row0, CPW)])


def kernel(src_token_ids_batch, tgt_token_ids_batch, src_table, tgt_table):
    pe = _pe_table().reshape(SEQ // CHUNK, CHUNK, EMBED_DIM)
    src_ids = src_token_ids_batch.astype(jnp.int32).reshape(ROWS // CHUNK, CHUNK)
    tgt_ids = tgt_token_ids_batch.astype(jnp.int32).reshape(ROWS // CHUNK, CHUNK)
    src_out, tgt_out = _sc_embed(src_ids, tgt_ids, src_table, tgt_table, pe)
    return (src_out.reshape(BATCH, SEQ, EMBED_DIM),
            tgt_out.reshape(BATCH, SEQ, EMBED_DIM))


# speedup vs baseline: 1.1112x; 1.0376x over previous
import functools
import math

import jax
import jax.numpy as jnp
from jax import lax
from jax.experimental import pallas as pl
from jax.experimental.pallas import tpu as pltpu, tpu_sc as plsc

EMBED_DIM = 128
SEQ = 2048
BATCH = 4
SCALE = math.sqrt(EMBED_DIM)

NW = 32
ROWS = BATCH * SEQ
RPW = ROWS // NW
CHUNK = 128
CPW = RPW // CHUNK
WPB = SEQ // RPW
L = 16


def _pe_table():
    position_id = jnp.arange(0, SEQ, dtype=jnp.float32)[:, None]
    frequencies = jnp.power(
        10000.0, -jnp.arange(0, EMBED_DIM, 2, dtype=jnp.float32) / EMBED_DIM)
    sin_part = jnp.sin(position_id * frequencies)
    cos_part = jnp.cos(position_id * frequencies)
    return jnp.stack([sin_part, cos_part], axis=-1).reshape(SEQ, EMBED_DIM)


@functools.partial(
    pl.kernel,
    mesh=plsc.VectorSubcoreMesh(core_axis_name="c", subcore_axis_name="s"),
    out_type=(
        jax.ShapeDtypeStruct((BATCH, SEQ, EMBED_DIM), jnp.float32),
        jax.ShapeDtypeStruct((BATCH, SEQ, EMBED_DIM), jnp.float32),
    ),
    scratch_types=[
        pltpu.VMEM((CPW, CHUNK), jnp.int32),
        pltpu.VMEM((CPW, CHUNK), jnp.int32),
        pltpu.VMEM((RPW, EMBED_DIM), jnp.float32),
        pltpu.VMEM((RPW, EMBED_DIM), jnp.float32),
        pltpu.VMEM((RPW, EMBED_DIM), jnp.float32),
        pltpu.SemaphoreType.DMA,
        pltpu.SemaphoreType.DMA,
    ],
)
def _sc_embed(src_ids, tgt_ids, src_tab, tgt_tab, pe,
              src_out, tgt_out,
              idx_s, idx_t, pe_v, rows_s, rows_t, sem_s, sem_t):
    cid = lax.axis_index("c")
    sid = lax.axis_index("s")
    wid = sid * 2 + cid
    b = wid // WPB
    s0 = lax.rem(wid, WPB) * RPW

    pltpu.sync_copy(src_ids.at[pl.ds(wid * CPW, CPW)], idx_s)
    pltpu.sync_copy(tgt_ids.at[pl.ds(wid * CPW, CPW)], idx_t)
    pltpu.sync_copy(pe.at[pl.ds(s0, RPW)], pe_v)

    cps = [pltpu.async_copy(src_tab.at[idx_s.at[j]],
                            rows_s.at[pl.ds(j * CHUNK, CHUNK)], sem_s)
           for j in range(CPW)]
    cpt = [pltpu.async_copy(tgt_tab.at[idx_t.at[j]],
                            rows_t.at[pl.ds(j * CHUNK, CHUNK)], sem_t)
           for j in range(CPW)]

    def fma_pass(rows_ref):
        def body(r, _):
            for l in range(EMBED_DIM // L):
                sl = pl.ds(l * L, L)
                rows_ref[r, sl] = rows_ref[r, sl] * SCALE + pe_v[r, sl]
            return 0
        lax.fori_loop(0, RPW, body, 0)

    for cp in cps:
        cp.wait()
    fma_pass(rows_s)
    st_s = pltpu.async_copy(rows_s, src_out.at[b, pl.ds(s0, RPW)], sem_s)

    for cp in cpt:
        cp.wait()
    fma_pass(rows_t)
    st_t = pltpu.async_copy(rows_t, tgt_out.at[b, pl.ds(s0, RPW)], sem_t)

    st_s.wait()
    st_t.wait()


def kernel(src_token_ids_batch, tgt_token_ids_batch, src_table, tgt_table):
    pe = _pe_table()
    src_ids = src_token_ids_batch.astype(jnp.int32).reshape(ROWS // CHUNK, CHUNK)
    tgt_ids = tgt_token_ids_batch.astype(jnp.int32).reshape(ROWS // CHUNK, CHUNK)
    return _sc_embed(src_ids, tgt_ids, src_table, tgt_table, pe)


# speedup vs baseline: 1.3568x; 1.2210x over previous
import functools
import math

import jax
import jax.numpy as jnp
import numpy as np
from jax import lax
from jax.experimental import pallas as pl
from jax.experimental.pallas import tpu as pltpu, tpu_sc as plsc

EMBED_DIM = 128
SEQ = 2048
BATCH = 4
SCALE = math.sqrt(EMBED_DIM)

NW = 32
ROWS = BATCH * SEQ
RPW = ROWS // NW
CHUNK = 128
CPW = RPW // CHUNK
WPB = SEQ // RPW
L = 16


def _pe_table():
    position_id = np.arange(0, SEQ, dtype=np.float32)[:, None]
    frequencies = np.power(
        np.float32(10000.0),
        -np.arange(0, EMBED_DIM, 2, dtype=np.float32) / np.float32(EMBED_DIM),
        dtype=np.float32)
    sin_part = np.sin(position_id * frequencies, dtype=np.float32)
    cos_part = np.cos(position_id * frequencies, dtype=np.float32)
    return np.stack([sin_part, cos_part], axis=-1).reshape(SEQ, EMBED_DIM)


_PE = _pe_table()


@functools.partial(
    pl.kernel,
    mesh=plsc.VectorSubcoreMesh(core_axis_name="c", subcore_axis_name="s"),
    out_type=(
        jax.ShapeDtypeStruct((BATCH, SEQ, EMBED_DIM), jnp.float32),
        jax.ShapeDtypeStruct((BATCH, SEQ, EMBED_DIM), jnp.float32),
    ),
    scratch_types=[
        pltpu.VMEM((RPW,), jnp.int32),
        pltpu.VMEM((RPW,), jnp.int32),
        pltpu.VMEM((RPW, EMBED_DIM), jnp.float32),
        pltpu.VMEM((RPW, EMBED_DIM), jnp.float32),
        pltpu.VMEM((RPW, EMBED_DIM), jnp.float32),
        pltpu.SemaphoreType.DMA,
        pltpu.SemaphoreType.DMA,
        pltpu.SemaphoreType.DMA,
    ],
)
def _sc_embed(src_ids, tgt_ids, src_tab, tgt_tab, pe,
              src_out, tgt_out,
              idx_s, idx_t, pe_v, rows_s, rows_t, sem_s, sem_t, sem_p):
    cid = lax.axis_index("c")
    sid = lax.axis_index("s")
    wid = sid * 2 + cid
    b = wid // WPB
    s0 = lax.rem(wid, WPB) * RPW

    cp_pe = pltpu.async_copy(pe.at[pl.ds(s0, RPW)], pe_v, sem_p)
    pltpu.sync_copy(src_ids.at[b, pl.ds(s0, RPW)], idx_s)
    cps = [pltpu.async_copy(src_tab.at[idx_s.at[pl.ds(j * CHUNK, CHUNK)]],
                            rows_s.at[pl.ds(j * CHUNK, CHUNK)], sem_s)
           for j in range(CPW)]
    pltpu.sync_copy(tgt_ids.at[b, pl.ds(s0, RPW)], idx_t)
    cpt = [pltpu.async_copy(tgt_tab.at[idx_t.at[pl.ds(j * CHUNK, CHUNK)]],
                            rows_t.at[pl.ds(j * CHUNK, CHUNK)], sem_t)
           for j in range(CPW)]

    def fma_pass(rows_ref):
        def body(r, _):
            for l in range(EMBED_DIM // L):
                sl = pl.ds(l * L, L)
                rows_ref[r, sl] = rows_ref[r, sl] * SCALE + pe_v[r, sl]
            return 0
        lax.fori_loop(0, RPW, body, 0)

    cp_pe.wait()
    for cp in cps:
        cp.wait()
    fma_pass(rows_s)
    st_s = pltpu.async_copy(rows_s, src_out.at[b, pl.ds(s0, RPW)], sem_s)

    for cp in cpt:
        cp.wait()
    fma_pass(rows_t)
    st_t = pltpu.async_copy(rows_t, tgt_out.at[b, pl.ds(s0, RPW)], sem_t)

    st_s.wait()
    st_t.wait()


def kernel(src_token_ids_batch, tgt_token_ids_batch, src_table, tgt_table):
    pe = jnp.asarray(_PE)
    src_ids = src_token_ids_batch.astype(jnp.int32)
    tgt_ids = tgt_token_ids_batch.astype(jnp.int32)
    return _sc_embed(src_ids, tgt_ids, src_table, tgt_table, pe)


# speedup vs baseline: 1.3783x; 1.0159x over previous
import functools
import math

import jax
import jax.numpy as jnp
import numpy as np
from jax import lax
from jax.experimental import pallas as pl
from jax.experimental.pallas import tpu as pltpu, tpu_sc as plsc

EMBED_DIM = 128
SEQ = 2048
BATCH = 4
SCALE = math.sqrt(EMBED_DIM)

NW = 32
ROWS = BATCH * SEQ
RPW = ROWS // NW
CHUNK = 128
CPW = RPW // CHUNK
WPB = SEQ // RPW
L = 16


def _pe_table():
    position_id = np.arange(0, SEQ, dtype=np.float32)[:, None]
    frequencies = np.power(
        np.float32(10000.0),
        -np.arange(0, EMBED_DIM, 2, dtype=np.float32) / np.float32(EMBED_DIM),
        dtype=np.float32)
    sin_part = np.sin(position_id * frequencies, dtype=np.float32)
    cos_part = np.cos(position_id * frequencies, dtype=np.float32)
    return np.stack([sin_part, cos_part], axis=-1).reshape(SEQ, EMBED_DIM)


_PE = _pe_table()


@functools.partial(
    pl.kernel,
    mesh=plsc.VectorSubcoreMesh(core_axis_name="c", subcore_axis_name="s"),
    out_type=(
        jax.ShapeDtypeStruct((BATCH, SEQ, EMBED_DIM), jnp.float32),
        jax.ShapeDtypeStruct((BATCH, SEQ, EMBED_DIM), jnp.float32),
    ),
    scratch_types=[
        pltpu.VMEM((RPW,), jnp.int32),
        pltpu.VMEM((RPW,), jnp.int32),
        pltpu.VMEM((RPW, EMBED_DIM), jnp.float32),
        pltpu.VMEM((RPW, EMBED_DIM), jnp.float32),
        pltpu.VMEM((RPW, EMBED_DIM), jnp.float32),
        pltpu.SemaphoreType.DMA,
        pltpu.SemaphoreType.DMA,
        pltpu.SemaphoreType.DMA,
        pltpu.SemaphoreType.DMA,
        pltpu.SemaphoreType.DMA,
        pltpu.SemaphoreType.DMA,
    ],
)
def _sc_embed(src_ids, tgt_ids, src_tab, tgt_tab, pe,
              src_out, tgt_out,
              idx_s, idx_t, pe_v, rows_s, rows_t,
              sem_s, sem_t, sem_p, sem_i, sem_os, sem_ot):
    cid = lax.axis_index("c")
    sid = lax.axis_index("s")
    wid = sid * 2 + cid
    b = wid // WPB
    s0 = lax.rem(wid, WPB) * RPW

    cp_pe = pltpu.async_copy(pe.at[pl.ds(s0, RPW)], pe_v, sem_p)
    cp_is = pltpu.async_copy(src_ids.at[b, pl.ds(s0, RPW)], idx_s, sem_i)
    cp_it = pltpu.async_copy(tgt_ids.at[b, pl.ds(s0, RPW)], idx_t, sem_i)
    cp_is.wait()
    cps = [pltpu.async_copy(src_tab.at[idx_s.at[pl.ds(j * CHUNK, CHUNK)]],
                            rows_s.at[pl.ds(j * CHUNK, CHUNK)], sem_s)
           for j in range(CPW)]
    cp_it.wait()
    cpt = [pltpu.async_copy(tgt_tab.at[idx_t.at[pl.ds(j * CHUNK, CHUNK)]],
                            rows_t.at[pl.ds(j * CHUNK, CHUNK)], sem_t)
           for j in range(CPW)]

    def fma_chunk(rows_ref, j):
        def body(r, _):
            for l in range(EMBED_DIM // L):
                sl = pl.ds(l * L, L)
                rows_ref[r, sl] = rows_ref[r, sl] * SCALE + pe_v[r, sl]
            return 0
        lax.fori_loop(j * CHUNK, (j + 1) * CHUNK, body, 0)

    cp_pe.wait()
    sts = []
    for j in range(CPW):
        cps[j].wait()
        fma_chunk(rows_s, j)
        sts.append(pltpu.async_copy(
            rows_s.at[pl.ds(j * CHUNK, CHUNK)],
            src_out.at[b, pl.ds(s0 + j * CHUNK, CHUNK)], sem_os))
    for j in range(CPW):
        cpt[j].wait()
        fma_chunk(rows_t, j)
        sts.append(pltpu.async_copy(
            rows_t.at[pl.ds(j * CHUNK, CHUNK)],
            tgt_out.at[b, pl.ds(s0 + j * CHUNK, CHUNK)], sem_ot))
    for st in sts:
        st.wait()


def kernel(src_token_ids_batch, tgt_token_ids_batch, src_table, tgt_table):
    pe = jnp.asarray(_PE)
    src_ids = src_token_ids_batch.astype(jnp.int32)
    tgt_ids = tgt_token_ids_batch.astype(jnp.int32)
    return _sc_embed(src_ids, tgt_ids, src_table, tgt_table, pe)
